# Initial kernel scaffold; baseline (speedup 1.0000x reference)
import jax
import jax.numpy as jnp
from jax.experimental import pallas as pl


def kernel(x, features, resolution):
    raise NotImplementedError("write your pallas kernel here")


# speedup vs baseline: 67.9157x; 67.9157x over previous
import functools

import numpy as np
import jax
import jax.numpy as jnp
from jax import lax
from jax.experimental import pallas as pl
from jax.experimental.pallas import tpu as pltpu
from jax.experimental.pallas import tpu_sc as plsc

N_LEVELS = 16
LOG2_T = 19
T = 2 ** LOG2_T
BASE_RES = 16
FINEST_RES = 512
N_POINTS = 1048576

_growth = np.exp((np.log(FINEST_RES) - np.log(BASE_RES)) / (N_LEVELS - 1))
RES = [int(BASE_RES * (_growth ** i)) for i in range(N_LEVELS)]

P1 = int(np.int32(np.uint32(2654435761)))
P2 = 805459861
MASK = T - 1

NW = 32
PPW = N_POINTS // NW
C = 512
VPC = C // 16
NCHUNK = PPW // C
W = 8
NROWS = N_LEVELS * T * 2 // W

_OFFS = [(i, j, k) for i in (0, 1) for j in (0, 1) for k in (0, 1)]

_mesh = plsc.VectorSubcoreMesh(core_axis_name="c", subcore_axis_name="s")


@functools.partial(
    pl.kernel,
    out_type=jax.ShapeDtypeStruct((N_POINTS * 2 * N_LEVELS,), jnp.float32),
    mesh=_mesh,
    compiler_params=pltpu.CompilerParams(
        needs_layout_passes=False, use_tc_tiling_on_sc=False),
    scratch_types=[
        pltpu.VMEM((C,), jnp.float32),
        pltpu.VMEM((C,), jnp.float32),
        pltpu.VMEM((C,), jnp.float32),
        pltpu.VMEM((3, C), jnp.float32),
        pltpu.VMEM((3, C), jnp.float32),
        pltpu.VMEM((8 * C,), jnp.int32),
        pltpu.VMEM((8 * C,), jnp.int32),
        pltpu.VMEM((8 * C,), jnp.int32),
        pltpu.VMEM((8 * C,), jnp.int32),
        pltpu.VMEM((8 * C, W), jnp.float32),
        pltpu.VMEM((8 * C, W), jnp.float32),
        pltpu.VMEM((C * 2 * N_LEVELS,), jnp.float32),
        pltpu.SemaphoreType.DMA,
        pltpu.SemaphoreType.DMA,
    ],
)
def _hashgrid(xt0, xt1, xt2, feat, out, xb0, xb1, xb2, wb0, wb1,
              idx0, idx1, col0, col1, rows0, rows1, ob, sem0, sem1):
    cid = lax.axis_index("c")
    sid = lax.axis_index("s")
    wid = sid * 2 + cid
    iota = lax.iota(jnp.int32, 16)

    wbufs = (wb0, wb1)
    ibufs = (idx0, idx1)
    cbufs = (col0, col1)
    rbufs = (rows0, rows1)
    sems = (sem0, sem1)

    def pass1(l, wb, ib, cb):
        rf = jnp.float32(RES[l])

        def vbody(v, carry):
            b = v * 16
            px = xb0[pl.ds(b, 16)] * rf
            py = xb1[pl.ds(b, 16)] * rf
            pz = xb2[pl.ds(b, 16)] * rf
            xi = px.astype(jnp.int32)
            yi = py.astype(jnp.int32)
            zi = pz.astype(jnp.int32)
            wb[0, pl.ds(b, 16)] = px - xi.astype(jnp.float32)
            wb[1, pl.ds(b, 16)] = py - yi.astype(jnp.float32)
            wb[2, pl.ds(b, 16)] = pz - zi.astype(jnp.float32)
            hx = (xi, xi + 1)
            hy0 = yi * P1
            hy = (hy0, hy0 + P1)
            hz0 = zi * P2
            hz = (hz0, hz0 + P2)
            lofs = l * T
            for ci, (i, j, k) in enumerate(_OFFS):
                g = ((hx[i] ^ hy[j] ^ hz[k]) & MASK) + lofs
                ib[pl.ds(ci * C + b, 16)] = lax.shift_right_logical(g, 2)
                cb[pl.ds(ci * C + b, 16)] = lax.shift_left(g & 3, 1)
            return carry

        lax.fori_loop(0, VPC, vbody, 0)

    def pass2(l, wb, cb, rb):
        def vbody(v, carry):
            b = v * 16
            wx = wb[0, pl.ds(b, 16)]
            wy = wb[1, pl.ds(b, 16)]
            wz = wb[2, pl.ds(b, 16)]
            prow = iota + b
            f = {}
            for ci, (i, j, k) in enumerate(_OFFS):
                ridx = prow + ci * C
                cv = cb[pl.ds(ci * C + b, 16)]
                for d in (0, 1):
                    f[(i, j, k, d)] = plsc.load_gather(rb, [ridx, cv + d])
            orow = prow * 32
            for d in (0, 1):
                g00 = f[(0, 0, 0, d)] + wz * (f[(0, 0, 1, d)] - f[(0, 0, 0, d)])
                g01 = f[(0, 1, 0, d)] + wz * (f[(0, 1, 1, d)] - f[(0, 1, 0, d)])
                g10 = f[(1, 0, 0, d)] + wz * (f[(1, 0, 1, d)] - f[(1, 0, 0, d)])
                g11 = f[(1, 1, 0, d)] + wz * (f[(1, 1, 1, d)] - f[(1, 1, 0, d)])
                h0 = g00 + wy * (g01 - g00)
                h1 = g10 + wy * (g11 - g10)
                r = h0 + wx * (h1 - h0)
                plsc.store_scatter(ob, [orow + (l * 2 + d)], r)
            return carry

        lax.fori_loop(0, VPC, vbody, 0)

    def chunk_body(ch, carry):
        gb = wid * PPW + ch * C
        pltpu.sync_copy(xt0.at[pl.ds(gb, C)], xb0)
        pltpu.sync_copy(xt1.at[pl.ds(gb, C)], xb1)
        pltpu.sync_copy(xt2.at[pl.ds(gb, C)], xb2)
        pass1(0, wbufs[0], ibufs[0], cbufs[0])
        pltpu.async_copy(feat.at[ibufs[0]], rbufs[0], sems[0])
        for l in range(N_LEVELS):
            p = l % 2
            if l + 1 < N_LEVELS:
                pn = (l + 1) % 2
                pass1(l + 1, wbufs[pn], ibufs[pn], cbufs[pn])
                pltpu.async_copy(feat.at[ibufs[pn]], rbufs[pn], sems[pn])
            pltpu.make_async_copy(feat.at[ibufs[p]], rbufs[p], sems[p]).wait()
            pass2(l, wbufs[p], cbufs[p], rbufs[p])
        pltpu.sync_copy(ob, out.at[pl.ds(gb * 32, C * 32)])
        return carry

    lax.fori_loop(0, NCHUNK, chunk_body, 0)


def kernel(x, features, resolution):
    del resolution
    ori_shape = x.shape[:-1]
    xt = x.reshape(-1, 3).T
    feat8 = features.reshape(NROWS, W)
    out = _hashgrid(xt[0], xt[1], xt[2], feat8)
    return out.reshape(*ori_shape, 2 * N_LEVELS)


# speedup vs baseline: 103.3247x; 1.5214x over previous
import functools

import numpy as np
import jax
import jax.numpy as jnp
from jax import lax
from jax.experimental import pallas as pl
from jax.experimental.pallas import tpu as pltpu
from jax.experimental.pallas import tpu_sc as plsc

N_LEVELS = 16
LOG2_T = 19
T = 2 ** LOG2_T
BASE_RES = 16
FINEST_RES = 512
N_POINTS = 1048576

_growth = np.exp((np.log(FINEST_RES) - np.log(BASE_RES)) / (N_LEVELS - 1))
RES = [int(BASE_RES * (_growth ** i)) for i in range(N_LEVELS)]

P1 = int(np.int32(np.uint32(2654435761)))
P2 = 805459861
MASK = T - 1

NW = 32
PPW = N_POINTS // NW
C = 256
VPC = C // 16
NCHUNK = PPW // C
W = 8
NROWS = N_LEVELS * T * 2 // W
LVL_STRIDE = T * 2 // W

_OFFS = [(i, j, k) for i in (0, 1) for j in (0, 1) for k in (0, 1)]

_mesh = plsc.VectorSubcoreMesh(core_axis_name="c", subcore_axis_name="s")


@functools.partial(
    pl.kernel,
    out_type=jax.ShapeDtypeStruct((N_POINTS * 2 * N_LEVELS,), jnp.float32),
    mesh=_mesh,
    compiler_params=pltpu.CompilerParams(
        needs_layout_passes=False, use_tc_tiling_on_sc=False),
    scratch_types=[
        pltpu.VMEM((C,), jnp.float32),
        pltpu.VMEM((C,), jnp.float32),
        pltpu.VMEM((C,), jnp.float32),
        pltpu.VMEM((3, C), jnp.float32),
        pltpu.VMEM((3, C), jnp.float32),
        pltpu.VMEM((16 * C,), jnp.int32),
        pltpu.VMEM((16 * C,), jnp.int32),
        pltpu.VMEM((8 * C,), jnp.int32),
        pltpu.VMEM((8 * C,), jnp.int32),
        pltpu.VMEM((16 * C, W), jnp.float32),
        pltpu.VMEM((16 * C, W), jnp.float32),
        pltpu.VMEM((C * 2 * N_LEVELS,), jnp.float32),
        pltpu.SemaphoreType.DMA,
        pltpu.SemaphoreType.DMA,
    ],
)
def _hashgrid(xt0, xt1, xt2, feat, out, xb0, xb1, xb2, wb0, wb1,
              idx0, idx1, col0, col1, rows0, rows1, ob, sem0, sem1):
    cid = lax.axis_index("c")
    sid = lax.axis_index("s")
    wid = sid * 2 + cid
    iota = lax.iota(jnp.int32, 16)

    wbufs = (wb0, wb1)
    ibufs = (idx0, idx1)
    cbufs = (col0, col1)
    rbufs = (rows0, rows1)
    sems = (sem0, sem1)

    def pass1(l, wb, ib, cb):
        rf = jnp.float32(RES[l])
        lofs = l * LVL_STRIDE

        def vbody(v, carry):
            b = v * 16
            px = xb0[pl.ds(b, 16)] * rf
            py = xb1[pl.ds(b, 16)] * rf
            pz = xb2[pl.ds(b, 16)] * rf
            xi = px.astype(jnp.int32)
            yi = py.astype(jnp.int32)
            zi = pz.astype(jnp.int32)
            wb[0, pl.ds(b, 16)] = px - xi.astype(jnp.float32)
            wb[1, pl.ds(b, 16)] = py - yi.astype(jnp.float32)
            wb[2, pl.ds(b, 16)] = pz - zi.astype(jnp.float32)
            hx = (xi, xi + 1)
            hy0 = yi * P1
            hy = (hy0, hy0 + P1)
            hz0 = zi * P2
            hz = (hz0, hz0 + P2)
            for ci, (i, j, k) in enumerate(_OFFS):
                h = (hx[i] ^ hy[j] ^ hz[k]) & MASK
                row0 = (lax.shift_right_logical(h, 2) & -32) \
                    + (lax.shift_right_logical(h, 3) & 15) + lofs
                ib[pl.ds((2 * ci) * C + b, 16)] = row0
                ib[pl.ds((2 * ci + 1) * C + b, 16)] = row0 + 16
                cb[pl.ds(ci * C + b, 16)] = h & 7
            return carry

        lax.fori_loop(0, VPC, vbody, 0)

    def pass2(l, wb, cb, rb):
        def vbody(v, carry):
            b = v * 16
            wx = wb[0, pl.ds(b, 16)]
            wy = wb[1, pl.ds(b, 16)]
            wz = wb[2, pl.ds(b, 16)]
            prow = iota + b
            f = {}
            for ci, (i, j, k) in enumerate(_OFFS):
                cv = cb[pl.ds(ci * C + b, 16)]
                for d in (0, 1):
                    f[(i, j, k, d)] = plsc.load_gather(
                        rb, [prow + (2 * ci + d) * C, cv])
            orow = prow * 32
            for d in (0, 1):
                g00 = f[(0, 0, 0, d)] + wz * (f[(0, 0, 1, d)] - f[(0, 0, 0, d)])
                g01 = f[(0, 1, 0, d)] + wz * (f[(0, 1, 1, d)] - f[(0, 1, 0, d)])
                g10 = f[(1, 0, 0, d)] + wz * (f[(1, 0, 1, d)] - f[(1, 0, 0, d)])
                g11 = f[(1, 1, 0, d)] + wz * (f[(1, 1, 1, d)] - f[(1, 1, 0, d)])
                h0 = g00 + wy * (g01 - g00)
                h1 = g10 + wy * (g11 - g10)
                r = h0 + wx * (h1 - h0)
                plsc.store_scatter(ob, [orow + (l * 2 + d)], r)
            return carry

        lax.fori_loop(0, VPC, vbody, 0)

    def chunk_body(ch, carry):
        gb = wid * PPW + ch * C
        pltpu.sync_copy(xt0.at[pl.ds(gb, C)], xb0)
        pltpu.sync_copy(xt1.at[pl.ds(gb, C)], xb1)
        pltpu.sync_copy(xt2.at[pl.ds(gb, C)], xb2)
        pass1(0, wbufs[0], ibufs[0], cbufs[0])
        pltpu.async_copy(feat.at[ibufs[0]], rbufs[0], sems[0])
        for l in range(N_LEVELS):
            p = l % 2
            if l + 1 < N_LEVELS:
                pn = (l + 1) % 2
                pass1(l + 1, wbufs[pn], ibufs[pn], cbufs[pn])
                pltpu.async_copy(feat.at[ibufs[pn]], rbufs[pn], sems[pn])
            pltpu.make_async_copy(feat.at[ibufs[p]], rbufs[p], sems[p]).wait()
            pass2(l, wbufs[p], cbufs[p], rbufs[p])
        pltpu.sync_copy(ob, out.at[pl.ds(gb * 32, C * 32)])
        return carry

    lax.fori_loop(0, NCHUNK, chunk_body, 0)


def kernel(x, features, resolution):
    del resolution
    ori_shape = x.shape[:-1]
    xt = x.reshape(-1, 3).T
    featv = jnp.transpose(
        features.reshape(N_LEVELS, T // 128, 128, 2), (0, 1, 3, 2)
    ).reshape(NROWS, W)
    out = _hashgrid(xt[0], xt[1], xt[2], featv)
    return out.reshape(*ori_shape, 2 * N_LEVELS)


# speedup vs baseline: 187.4392x; 1.8141x over previous
import functools

import numpy as np
import jax
import jax.numpy as jnp
from jax import lax
from jax.experimental import pallas as pl
from jax.experimental.pallas import tpu as pltpu
from jax.experimental.pallas import tpu_sc as plsc

N_LEVELS = 16
LOG2_T = 19
T = 2 ** LOG2_T
BASE_RES = 16
FINEST_RES = 512
N_POINTS = 1048576

_growth = np.exp((np.log(FINEST_RES) - np.log(BASE_RES)) / (N_LEVELS - 1))
RES = [int(BASE_RES * (_growth ** i)) for i in range(N_LEVELS)]

P1 = int(np.int32(np.uint32(2654435761)))
P2 = 805459861
MASK = T - 1

NW = 32
NS = 16
PPW = N_POINTS // NW
C = 512
VPC = C // 16
NCHUNK = PPW // C
W = 8
NROWS = N_LEVELS * T * 2 // W
LVL_STRIDE = T // 4

NWORDS = N_LEVELS * T * 2
TPW = NWORDS // NS
CVW = 4096
NCV = TPW // CVW

_OFFS = [(i, j, k) for i in (0, 1) for j in (0, 1) for k in (0, 1)]

_mesh = plsc.VectorSubcoreMesh(core_axis_name="c", subcore_axis_name="s")


@functools.partial(
    pl.kernel,
    out_type=(
        jax.ShapeDtypeStruct((N_POINTS * 2 * N_LEVELS,), jnp.float32),
        jax.ShapeDtypeStruct((2 * NROWS, W), jnp.float32),
    ),
    mesh=_mesh,
    compiler_params=pltpu.CompilerParams(
        needs_layout_passes=False, use_tc_tiling_on_sc=False),
    scratch_types=[
        pltpu.VMEM((C,), jnp.float32),
        pltpu.VMEM((C,), jnp.float32),
        pltpu.VMEM((C,), jnp.float32),
        pltpu.VMEM((3, C), jnp.float32),
        pltpu.VMEM((3, C), jnp.float32),
        pltpu.VMEM((8 * C,), jnp.int32),
        pltpu.VMEM((8 * C,), jnp.int32),
        pltpu.VMEM((8 * C,), jnp.int32),
        pltpu.VMEM((8 * C,), jnp.int32),
        pltpu.VMEM((8 * C, W), jnp.float32),
        pltpu.VMEM((8 * C, W), jnp.float32),
        pltpu.VMEM((C * 2 * N_LEVELS,), jnp.float32),
        pltpu.VMEM((CVW,), jnp.float32),
        pltpu.VMEM((CVW // W, W), jnp.float32),
        pltpu.SemaphoreType.DMA,
        pltpu.SemaphoreType.DMA,
    ],
)
def _hashgrid(xt0, xt1, xt2, feat, out, conv, xb0, xb1, xb2, wb0, wb1,
              idx0, idx1, col0, col1, rows0, rows1, ob, pin, pout,
              sem0, sem1):
    cid = lax.axis_index("c")
    sid = lax.axis_index("s")
    wid = sid * 2 + cid
    iota = lax.iota(jnp.int32, 16)
    iota2 = iota * 2
    rowpat = lax.shift_right_logical(iota2, 3)
    colpat = iota2 & 7

    cbase = cid * NROWS

    def conv_chunk(cc, carry):
        src = sid * TPW + cc * CVW
        dst_row = cbase + sid * (TPW // W) + cc * (CVW // W)
        pltpu.sync_copy(feat.at[pl.ds(src, CVW)], pin)
        for u in range(CVW // 256):
            for g in range(8):
                v0 = pin[pl.ds(u * 256 + g * 16, 16)]
                v1 = pin[pl.ds(u * 256 + 128 + g * 16, 16)]
                rowv = rowpat + (u * 32 + g * 4)
                plsc.store_scatter(pout, [rowv, colpat], v0)
                plsc.store_scatter(pout, [rowv, colpat + 1], v1)
        pltpu.sync_copy(pout, conv.at[pl.ds(dst_row, CVW // W), :])
        return carry

    lax.fori_loop(0, NCV, conv_chunk, 0)
    plsc.subcore_barrier()

    wbufs = (wb0, wb1)
    ibufs = (idx0, idx1)
    cbufs = (col0, col1)
    rbufs = (rows0, rows1)
    sems = (sem0, sem1)

    def pass1(l, wb, ib, cb):
        rf = jnp.float32(RES[l])
        lofs = cbase + l * LVL_STRIDE

        def vbody(v, carry):
            b = v * 16
            px = xb0[pl.ds(b, 16)] * rf
            py = xb1[pl.ds(b, 16)] * rf
            pz = xb2[pl.ds(b, 16)] * rf
            xi = px.astype(jnp.int32)
            yi = py.astype(jnp.int32)
            zi = pz.astype(jnp.int32)
            wb[0, pl.ds(b, 16)] = px - xi.astype(jnp.float32)
            wb[1, pl.ds(b, 16)] = py - yi.astype(jnp.float32)
            wb[2, pl.ds(b, 16)] = pz - zi.astype(jnp.float32)
            hx = (xi, xi + 1)
            hy0 = yi * P1
            hy = (hy0, hy0 + P1)
            hz0 = zi * P2
            hz = (hz0, hz0 + P2)
            for ci, (i, j, k) in enumerate(_OFFS):
                h = (hx[i] ^ hy[j] ^ hz[k]) & MASK
                ib[pl.ds(ci * C + b, 16)] = \
                    lax.shift_right_logical(h, 2) + lofs
                cb[pl.ds(ci * C + b, 16)] = lax.shift_left(h & 3, 1)
            return carry

        lax.fori_loop(0, VPC, vbody, 0)

    def pass2(l, wb, cb, rb):
        def vbody(v, carry):
            b = v * 16
            wx = wb[0, pl.ds(b, 16)]
            wy = wb[1, pl.ds(b, 16)]
            wz = wb[2, pl.ds(b, 16)]
            prow = iota + b
            f = {}
            for ci, (i, j, k) in enumerate(_OFFS):
                cv = cb[pl.ds(ci * C + b, 16)]
                ridx = prow + ci * C
                for d in (0, 1):
                    f[(i, j, k, d)] = plsc.load_gather(rb, [ridx, cv + d])
            orow = prow * 32
            for d in (0, 1):
                g00 = f[(0, 0, 0, d)] + wz * (f[(0, 0, 1, d)] - f[(0, 0, 0, d)])
                g01 = f[(0, 1, 0, d)] + wz * (f[(0, 1, 1, d)] - f[(0, 1, 0, d)])
                g10 = f[(1, 0, 0, d)] + wz * (f[(1, 0, 1, d)] - f[(1, 0, 0, d)])
                g11 = f[(1, 1, 0, d)] + wz * (f[(1, 1, 1, d)] - f[(1, 1, 0, d)])
                h0 = g00 + wy * (g01 - g00)
                h1 = g10 + wy * (g11 - g10)
                r = h0 + wx * (h1 - h0)
                plsc.store_scatter(ob, [orow + (l * 2 + d)], r)
            return carry

        lax.fori_loop(0, VPC, vbody, 0)

    def chunk_body(ch, carry):
        gb = wid * PPW + ch * C
        pltpu.sync_copy(xt0.at[pl.ds(gb, C)], xb0)
        pltpu.sync_copy(xt1.at[pl.ds(gb, C)], xb1)
        pltpu.sync_copy(xt2.at[pl.ds(gb, C)], xb2)
        pass1(0, wbufs[0], ibufs[0], cbufs[0])
        pltpu.async_copy(conv.at[ibufs[0]], rbufs[0], sems[0])
        for l in range(N_LEVELS):
            p = l % 2
            if l + 1 < N_LEVELS:
                pn = (l + 1) % 2
                pass1(l + 1, wbufs[pn], ibufs[pn], cbufs[pn])
                pltpu.async_copy(conv.at[ibufs[pn]], rbufs[pn], sems[pn])
            pltpu.make_async_copy(conv.at[ibufs[p]], rbufs[p], sems[p]).wait()
            pass2(l, wbufs[p], cbufs[p], rbufs[p])
        pltpu.sync_copy(ob, out.at[pl.ds(gb * 32, C * 32)])
        return carry

    lax.fori_loop(0, NCHUNK, chunk_body, 0)


def kernel(x, features, resolution):
    del resolution
    ori_shape = x.shape[:-1]
    xt = x.reshape(-1, 3).T
    featv = jnp.transpose(
        features.reshape(N_LEVELS, T // 128, 128, 2), (0, 1, 3, 2)
    ).reshape(NWORDS)
    out, _ = _hashgrid(xt[0], xt[1], xt[2], featv)
    return out.reshape(*ori_shape, 2 * N_LEVELS)


# speedup vs baseline: 202.2916x; 1.0792x over previous
import functools

import numpy as np
import jax
import jax.numpy as jnp
from jax import lax
from jax.experimental import pallas as pl
from jax.experimental.pallas import tpu as pltpu
from jax.experimental.pallas import tpu_sc as plsc

N_LEVELS = 16
LOG2_T = 19
T = 2 ** LOG2_T
BASE_RES = 16
FINEST_RES = 512
N_POINTS = 1048576

_growth = np.exp((np.log(FINEST_RES) - np.log(BASE_RES)) / (N_LEVELS - 1))
RES = [int(BASE_RES * (_growth ** i)) for i in range(N_LEVELS)]

P1 = int(np.int32(np.uint32(2654435761)))
P2 = 805459861
MASK = T - 1

NW = 32
NS = 16
PPW = N_POINTS // NW
C = 256
VPC = C // 16
NCHUNK = PPW // C
W = 8
NROWS = N_LEVELS * T * 2 // W
LVL_STRIDE = T // 4

NWORDS = N_LEVELS * T * 2
TPW = NWORDS // NS
CVW = 4096
NCV = TPW // CVW

_OFFS = [(i, j, k) for i in (0, 1) for j in (0, 1) for k in (0, 1)]

N_DENSE = 3
_DS = [RES[l] + 1 for l in range(N_DENSE)]
_DN = [s * s * s for s in _DS]
DCHUNK = 2048

_mesh = plsc.VectorSubcoreMesh(core_axis_name="c", subcore_axis_name="s")


@functools.partial(
    pl.kernel,
    out_type=(
        jax.ShapeDtypeStruct((N_POINTS * 2 * N_LEVELS,), jnp.float32),
        jax.ShapeDtypeStruct((2 * NROWS, W), jnp.float32),
    ),
    mesh=_mesh,
    compiler_params=pltpu.CompilerParams(
        needs_layout_passes=False, use_tc_tiling_on_sc=False),
    scratch_types=[
        pltpu.VMEM((C,), jnp.float32),
        pltpu.VMEM((C,), jnp.float32),
        pltpu.VMEM((C,), jnp.float32),
        pltpu.VMEM((3, C), jnp.float32),
        pltpu.VMEM((3, C), jnp.float32),
        pltpu.VMEM((8 * C,), jnp.int32),
        pltpu.VMEM((8 * C,), jnp.int32),
        pltpu.VMEM((8 * C,), jnp.int32),
        pltpu.VMEM((8 * C,), jnp.int32),
        pltpu.VMEM((8 * C, W), jnp.float32),
        pltpu.VMEM((8 * C, W), jnp.float32),
        pltpu.VMEM((C * 2 * N_LEVELS,), jnp.float32),
        pltpu.VMEM((CVW,), jnp.float32),
        pltpu.VMEM((CVW // W, W), jnp.float32),
        pltpu.VMEM((_DN[0] * 2,), jnp.float32),
        pltpu.VMEM((_DN[1] * 2,), jnp.float32),
        pltpu.VMEM((_DN[2] * 2,), jnp.float32),
        pltpu.SemaphoreType.DMA,
        pltpu.SemaphoreType.DMA,
    ],
)
def _hashgrid(xt0, xt1, xt2, feat, out, conv, xb0, xb1, xb2, wb0, wb1,
              idx0, idx1, col0, col1, rows0, rows1, ob, pin, pout,
              dense0, dense1, dense2, sem0, sem1):
    cid = lax.axis_index("c")
    sid = lax.axis_index("s")
    wid = sid * 2 + cid
    iota = lax.iota(jnp.int32, 16)
    iota2 = iota * 2
    rowpat = lax.shift_right_logical(iota2, 3)
    colpat = iota2 & 7

    cbase = cid * NROWS

    def conv_chunk(cc, carry):
        src = sid * TPW + cc * CVW
        dst_row = cbase + sid * (TPW // W) + cc * (CVW // W)
        pltpu.sync_copy(feat.at[pl.ds(src, CVW)], pin)
        for u in range(CVW // 256):
            for g in range(8):
                v0 = pin[pl.ds(u * 256 + g * 16, 16)]
                v1 = pin[pl.ds(u * 256 + 128 + g * 16, 16)]
                rowv = rowpat + (u * 32 + g * 4)
                plsc.store_scatter(pout, [rowv, colpat], v0)
                plsc.store_scatter(pout, [rowv, colpat + 1], v1)
        pltpu.sync_copy(pout, conv.at[pl.ds(dst_row, CVW // W), :])
        return carry

    lax.fori_loop(0, NCV, conv_chunk, 0)
    plsc.subcore_barrier()

    denses = (dense0, dense1, dense2)
    for l in range(N_DENSE):
        S = _DS[l]
        NE = _DN[l]
        lofs = cbase + l * LVL_STRIDE
        inv2 = float(1.0 / (S * S))
        inv1 = float(1.0 / S)
        dense = denses[l]
        nch = -(-NE // DCHUNK)

        def dbody(chv, carry, S=S, NE=NE, lofs=lofs, inv2=inv2, inv1=inv1,
                  dense=dense):
            e0 = chv * DCHUNK

            def gen(v, c2):
                e = jnp.minimum(iota + (e0 + v * 16), NE - 1)
                xg = ((e.astype(jnp.float32) + 0.5) * inv2).astype(jnp.int32)
                r = e - xg * (S * S)
                yg = ((r.astype(jnp.float32) + 0.5) * inv1).astype(jnp.int32)
                zg = r - yg * S
                h = (xg ^ (yg * P1) ^ (zg * P2)) & MASK
                idx0[pl.ds(v * 16, 16)] = lax.shift_right_logical(h, 2) + lofs
                col0[pl.ds(v * 16, 16)] = lax.shift_left(h & 3, 1)
                return c2

            lax.fori_loop(0, DCHUNK // 16, gen, 0)
            pltpu.async_copy(conv.at[idx0], rows0, sem0).wait()

            def rep(v, c2):
                b = v * 16
                cv = col0[pl.ds(b, 16)]
                prow = iota + b
                dpos = jnp.minimum(iota + (e0 + b), NE - 1) * 2
                d0v = plsc.load_gather(rows0, [prow, cv])
                d1v = plsc.load_gather(rows0, [prow, cv + 1])
                plsc.store_scatter(dense, [dpos], d0v)
                plsc.store_scatter(dense, [dpos + 1], d1v)
                return c2

            lax.fori_loop(0, DCHUNK // 16, rep, 0)
            return carry

        lax.fori_loop(0, nch, dbody, 0)

    wbufs = (wb0, wb1)
    ibufs = (idx0, idx1)
    cbufs = (col0, col1)
    rbufs = (rows0, rows1)
    sems = (sem0, sem1)

    def pass1(l, wb, ib, cb):
        rf = jnp.float32(RES[l])
        lofs = cbase + l * LVL_STRIDE

        def vbody(v, carry):
            b = v * 16
            px = xb0[pl.ds(b, 16)] * rf
            py = xb1[pl.ds(b, 16)] * rf
            pz = xb2[pl.ds(b, 16)] * rf
            xi = px.astype(jnp.int32)
            yi = py.astype(jnp.int32)
            zi = pz.astype(jnp.int32)
            wb[0, pl.ds(b, 16)] = px - xi.astype(jnp.float32)
            wb[1, pl.ds(b, 16)] = py - yi.astype(jnp.float32)
            wb[2, pl.ds(b, 16)] = pz - zi.astype(jnp.float32)
            hx = (xi, xi + 1)
            hy0 = yi * P1
            hy = (hy0, hy0 + P1)
            hz0 = zi * P2
            hz = (hz0, hz0 + P2)
            for ci, (i, j, k) in enumerate(_OFFS):
                h = (hx[i] ^ hy[j] ^ hz[k]) & MASK
                ib[pl.ds(ci * C + b, 16)] = \
                    lax.shift_right_logical(h, 2) + lofs
                cb[pl.ds(ci * C + b, 16)] = lax.shift_left(h & 3, 1)
            return carry

        lax.fori_loop(0, VPC, vbody, 0)

    def pass2(l, wb, cb, rb):
        def vbody(v, carry):
            b = v * 16
            wx = wb[0, pl.ds(b, 16)]
            wy = wb[1, pl.ds(b, 16)]
            wz = wb[2, pl.ds(b, 16)]
            prow = iota + b
            f = {}
            for ci, (i, j, k) in enumerate(_OFFS):
                cv = cb[pl.ds(ci * C + b, 16)]
                ridx = prow + ci * C
                for d in (0, 1):
                    f[(i, j, k, d)] = plsc.load_gather(rb, [ridx, cv + d])
            orow = prow * 32
            for d in (0, 1):
                g00 = f[(0, 0, 0, d)] + wz * (f[(0, 0, 1, d)] - f[(0, 0, 0, d)])
                g01 = f[(0, 1, 0, d)] + wz * (f[(0, 1, 1, d)] - f[(0, 1, 0, d)])
                g10 = f[(1, 0, 0, d)] + wz * (f[(1, 0, 1, d)] - f[(1, 0, 0, d)])
                g11 = f[(1, 1, 0, d)] + wz * (f[(1, 1, 1, d)] - f[(1, 1, 0, d)])
                h0 = g00 + wy * (g01 - g00)
                h1 = g10 + wy * (g11 - g10)
                r = h0 + wx * (h1 - h0)
                plsc.store_scatter(ob, [orow + (l * 2 + d)], r)
            return carry

        lax.fori_loop(0, VPC, vbody, 0)

    def dense_level(l):
        S = _DS[l]
        rf = jnp.float32(RES[l])
        dense = denses[l]

        def vbody(v, carry):
            b = v * 16
            px = xb0[pl.ds(b, 16)] * rf
            py = xb1[pl.ds(b, 16)] * rf
            pz = xb2[pl.ds(b, 16)] * rf
            xi = px.astype(jnp.int32)
            yi = py.astype(jnp.int32)
            zi = pz.astype(jnp.int32)
            wx = px - xi.astype(jnp.float32)
            wy = py - yi.astype(jnp.float32)
            wz = pz - zi.astype(jnp.float32)
            base2 = ((xi * S + yi) * S + zi) * 2
            f = {}
            for (i, j, k) in _OFFS:
                dv = base2 + ((i * S + j) * S + k) * 2
                for d in (0, 1):
                    f[(i, j, k, d)] = plsc.load_gather(dense, [dv + d])
            orow = (iota + b) * 32
            for d in (0, 1):
                g00 = f[(0, 0, 0, d)] + wz * (f[(0, 0, 1, d)] - f[(0, 0, 0, d)])
                g01 = f[(0, 1, 0, d)] + wz * (f[(0, 1, 1, d)] - f[(0, 1, 0, d)])
                g10 = f[(1, 0, 0, d)] + wz * (f[(1, 0, 1, d)] - f[(1, 0, 0, d)])
                g11 = f[(1, 1, 0, d)] + wz * (f[(1, 1, 1, d)] - f[(1, 1, 0, d)])
                h0 = g00 + wy * (g01 - g00)
                h1 = g10 + wy * (g11 - g10)
                r = h0 + wx * (h1 - h0)
                plsc.store_scatter(ob, [orow + (l * 2 + d)], r)
            return carry

        lax.fori_loop(0, VPC, vbody, 0)

    SLV = list(range(N_DENSE, N_LEVELS))

    def chunk_body(ch, carry):
        gb = wid * PPW + ch * C
        pltpu.sync_copy(xt0.at[pl.ds(gb, C)], xb0)
        pltpu.sync_copy(xt1.at[pl.ds(gb, C)], xb1)
        pltpu.sync_copy(xt2.at[pl.ds(gb, C)], xb2)
        pass1(SLV[0], wbufs[0], ibufs[0], cbufs[0])
        pltpu.async_copy(conv.at[ibufs[0]], rbufs[0], sems[0])
        for l in range(N_DENSE):
            dense_level(l)
        for t, l in enumerate(SLV):
            p = t % 2
            if t + 1 < len(SLV):
                pn = (t + 1) % 2
                pass1(SLV[t + 1], wbufs[pn], ibufs[pn], cbufs[pn])
                pltpu.async_copy(conv.at[ibufs[pn]], rbufs[pn], sems[pn])
            pltpu.make_async_copy(conv.at[ibufs[p]], rbufs[p], sems[p]).wait()
            pass2(l, wbufs[p], cbufs[p], rbufs[p])
        pltpu.sync_copy(ob, out.at[pl.ds(gb * 32, C * 32)])
        return carry

    lax.fori_loop(0, NCHUNK, chunk_body, 0)


def kernel(x, features, resolution):
    del resolution
    ori_shape = x.shape[:-1]
    xt = x.reshape(-1, 3).T
    featv = jnp.transpose(
        features.reshape(N_LEVELS, T // 128, 128, 2), (0, 1, 3, 2)
    ).reshape(NWORDS)
    out, _ = _hashgrid(xt[0], xt[1], xt[2], featv)
    return out.reshape(*ori_shape, 2 * N_LEVELS)


# speedup vs baseline: 212.0623x; 1.0483x over previous
import functools

import numpy as np
import jax
import jax.numpy as jnp
from jax import lax
from jax.experimental import pallas as pl
from jax.experimental.pallas import tpu as pltpu
from jax.experimental.pallas import tpu_sc as plsc

N_LEVELS = 16
LOG2_T = 19
T = 2 ** LOG2_T
BASE_RES = 16
FINEST_RES = 512
N_POINTS = 1048576

_growth = np.exp((np.log(FINEST_RES) - np.log(BASE_RES)) / (N_LEVELS - 1))
RES = [int(BASE_RES * (_growth ** i)) for i in range(N_LEVELS)]

P1 = int(np.int32(np.uint32(2654435761)))
P2 = 805459861
MASK = T - 1

NW = 32
NS = 16
PPW = N_POINTS // NW
C = 256
VPC = C // 16
NCHUNK = PPW // C
W = 8
NROWS = N_LEVELS * T * 2 // W
LVL_STRIDE = T // 4

NWORDS = N_LEVELS * T * 2
TPW = NWORDS // NS
CVW = 4096
NCV = TPW // CVW

_OFFS = [(i, j, k) for i in (0, 1) for j in (0, 1) for k in (0, 1)]

N_DENSE = 3
_DS = [RES[l] + 1 for l in range(N_DENSE)]
_DN = [s * s * s for s in _DS]
DCHUNK = 2048

_mesh = plsc.VectorSubcoreMesh(core_axis_name="c", subcore_axis_name="s")


@functools.partial(
    pl.kernel,
    out_type=(
        jax.ShapeDtypeStruct((N_POINTS * 2 * N_LEVELS,), jnp.float32),
        jax.ShapeDtypeStruct((2 * NROWS, W), jnp.float32),
    ),
    mesh=_mesh,
    compiler_params=pltpu.CompilerParams(
        needs_layout_passes=False, use_tc_tiling_on_sc=False),
    scratch_types=[
        pltpu.VMEM((C,), jnp.float32),
        pltpu.VMEM((C,), jnp.float32),
        pltpu.VMEM((C,), jnp.float32),
        pltpu.VMEM((3, C), jnp.float32),
        pltpu.VMEM((3, C), jnp.float32),
        pltpu.VMEM((8 * C,), jnp.int32),
        pltpu.VMEM((8 * C,), jnp.int32),
        pltpu.VMEM((8 * C,), jnp.int32),
        pltpu.VMEM((8 * C,), jnp.int32),
        pltpu.VMEM((8 * C, W), jnp.float32),
        pltpu.VMEM((8 * C, W), jnp.float32),
        pltpu.VMEM((C * 2 * N_LEVELS,), jnp.float32),
        pltpu.VMEM((CVW,), jnp.float32),
        pltpu.VMEM((CVW // W, W), jnp.float32),
        pltpu.VMEM((_DN[0] * 2,), jnp.float32),
        pltpu.VMEM((_DN[1] * 2,), jnp.float32),
        pltpu.VMEM((_DN[2] * 2,), jnp.float32),
        pltpu.SemaphoreType.DMA,
        pltpu.SemaphoreType.DMA,
        pltpu.SemaphoreType.DMA,
        pltpu.SemaphoreType.DMA,
    ],
)
def _hashgrid(xt0, xt1, xt2, feat, out, conv, xb0, xb1, xb2, wb0, wb1,
              idx0, idx1, col0, col1, rows0, rows1, ob, pin, pout,
              dense0, dense1, dense2, sem0, sem1, sem2, sem3):
    cid = lax.axis_index("c")
    sid = lax.axis_index("s")
    wid = sid * 2 + cid
    iota = lax.iota(jnp.int32, 16)
    iota2 = iota * 2
    rowpat = lax.shift_right_logical(iota2, 3)
    colpat = iota2 & 7

    cbase = cid * NROWS

    def conv_chunk(cc, carry):
        src = sid * TPW + cc * CVW
        dst_row = cbase + sid * (TPW // W) + cc * (CVW // W)
        pltpu.sync_copy(feat.at[pl.ds(src, CVW)], pin)
        for u in range(CVW // 256):
            for g in range(8):
                v0 = pin[pl.ds(u * 256 + g * 16, 16)]
                v1 = pin[pl.ds(u * 256 + 128 + g * 16, 16)]
                rowv = rowpat + (u * 32 + g * 4)
                plsc.store_scatter(pout, [rowv, colpat], v0)
                plsc.store_scatter(pout, [rowv, colpat + 1], v1)
        pltpu.sync_copy(pout, conv.at[pl.ds(dst_row, CVW // W), :])
        return carry

    lax.fori_loop(0, NCV, conv_chunk, 0)
    plsc.subcore_barrier()

    denses = (dense0, dense1, dense2)
    for l in range(N_DENSE):
        S = _DS[l]
        NE = _DN[l]
        lofs = cbase + l * LVL_STRIDE
        inv2 = float(1.0 / (S * S))
        inv1 = float(1.0 / S)
        dense = denses[l]
        nch = -(-NE // DCHUNK)

        def dbody(chv, carry, S=S, NE=NE, lofs=lofs, inv2=inv2, inv1=inv1,
                  dense=dense):
            e0 = chv * DCHUNK

            def gen(v, c2):
                e = jnp.minimum(iota + (e0 + v * 16), NE - 1)
                xg = ((e.astype(jnp.float32) + 0.5) * inv2).astype(jnp.int32)
                r = e - xg * (S * S)
                yg = ((r.astype(jnp.float32) + 0.5) * inv1).astype(jnp.int32)
                zg = r - yg * S
                h = (xg ^ (yg * P1) ^ (zg * P2)) & MASK
                idx0[pl.ds(v * 16, 16)] = lax.shift_right_logical(h, 2) + lofs
                col0[pl.ds(v * 16, 16)] = lax.shift_left(h & 3, 1)
                return c2

            lax.fori_loop(0, DCHUNK // 16, gen, 0)
            pltpu.async_copy(conv.at[idx0], rows0, sem0).wait()

            def rep(v, c2):
                b = v * 16
                cv = col0[pl.ds(b, 16)]
                prow = iota + b
                dpos = jnp.minimum(iota + (e0 + b), NE - 1) * 2
                d0v = plsc.load_gather(rows0, [prow, cv])
                d1v = plsc.load_gather(rows0, [prow, cv + 1])
                plsc.store_scatter(dense, [dpos], d0v)
                plsc.store_scatter(dense, [dpos + 1], d1v)
                return c2

            lax.fori_loop(0, DCHUNK // 16, rep, 0)
            return carry

        lax.fori_loop(0, nch, dbody, 0)

    wbufs = (wb0, wb1)
    ibufs = (idx0, idx1)
    cbufs = (col0, col1)
    rbufs = (rows0, rows1)
    sems = ((sem0, sem1), (sem2, sem3))
    HG = 4 * C

    def fire(p):
        ib, rb = ibufs[p], rbufs[p]
        pltpu.async_copy(conv.at[ib.at[pl.ds(0, HG)]],
                         rb.at[pl.ds(0, HG), :], sems[p][0])
        pltpu.async_copy(conv.at[ib.at[pl.ds(HG, HG)]],
                         rb.at[pl.ds(HG, HG), :], sems[p][1])

    def drain(p):
        ib, rb = ibufs[p], rbufs[p]
        pltpu.make_async_copy(conv.at[ib.at[pl.ds(0, HG)]],
                              rb.at[pl.ds(0, HG), :], sems[p][0]).wait()
        pltpu.make_async_copy(conv.at[ib.at[pl.ds(HG, HG)]],
                              rb.at[pl.ds(HG, HG), :], sems[p][1]).wait()

    def pass1(l, wb, ib, cb):
        rf = jnp.float32(RES[l])
        lofs = cbase + l * LVL_STRIDE

        def vbody(v, carry):
            b = v * 16
            px = xb0[pl.ds(b, 16)] * rf
            py = xb1[pl.ds(b, 16)] * rf
            pz = xb2[pl.ds(b, 16)] * rf
            xi = px.astype(jnp.int32)
            yi = py.astype(jnp.int32)
            zi = pz.astype(jnp.int32)
            wb[0, pl.ds(b, 16)] = px - xi.astype(jnp.float32)
            wb[1, pl.ds(b, 16)] = py - yi.astype(jnp.float32)
            wb[2, pl.ds(b, 16)] = pz - zi.astype(jnp.float32)
            hx = (xi, xi + 1)
            hy0 = yi * P1
            hy = (hy0, hy0 + P1)
            hz0 = zi * P2
            hz = (hz0, hz0 + P2)
            for ci, (i, j, k) in enumerate(_OFFS):
                h = (hx[i] ^ hy[j] ^ hz[k]) & MASK
                ib[pl.ds(ci * C + b, 16)] = \
                    lax.shift_right_logical(h, 2) + lofs
                cb[pl.ds(ci * C + b, 16)] = lax.shift_left(h & 3, 1)
            return carry

        lax.fori_loop(0, VPC, vbody, 0)

    def pass2(l, wb, cb, rb):
        def vbody(v, carry):
            b = v * 16
            wx = wb[0, pl.ds(b, 16)]
            wy = wb[1, pl.ds(b, 16)]
            wz = wb[2, pl.ds(b, 16)]
            prow = iota + b
            f = {}
            for ci, (i, j, k) in enumerate(_OFFS):
                cv = cb[pl.ds(ci * C + b, 16)]
                ridx = prow + ci * C
                for d in (0, 1):
                    f[(i, j, k, d)] = plsc.load_gather(rb, [ridx, cv + d])
            orow = prow * 32
            for d in (0, 1):
                g00 = f[(0, 0, 0, d)] + wz * (f[(0, 0, 1, d)] - f[(0, 0, 0, d)])
                g01 = f[(0, 1, 0, d)] + wz * (f[(0, 1, 1, d)] - f[(0, 1, 0, d)])
                g10 = f[(1, 0, 0, d)] + wz * (f[(1, 0, 1, d)] - f[(1, 0, 0, d)])
                g11 = f[(1, 1, 0, d)] + wz * (f[(1, 1, 1, d)] - f[(1, 1, 0, d)])
                h0 = g00 + wy * (g01 - g00)
                h1 = g10 + wy * (g11 - g10)
                r = h0 + wx * (h1 - h0)
                plsc.store_scatter(ob, [orow + (l * 2 + d)], r)
            return carry

        lax.fori_loop(0, VPC, vbody, 0)

    def dense_level(l):
        S = _DS[l]
        rf = jnp.float32(RES[l])
        dense = denses[l]

        def vbody(v, carry):
            b = v * 16
            px = xb0[pl.ds(b, 16)] * rf
            py = xb1[pl.ds(b, 16)] * rf
            pz = xb2[pl.ds(b, 16)] * rf
            xi = px.astype(jnp.int32)
            yi = py.astype(jnp.int32)
            zi = pz.astype(jnp.int32)
            wx = px - xi.astype(jnp.float32)
            wy = py - yi.astype(jnp.float32)
            wz = pz - zi.astype(jnp.float32)
            base2 = ((xi * S + yi) * S + zi) * 2
            f = {}
            for (i, j, k) in _OFFS:
                dv = base2 + ((i * S + j) * S + k) * 2
                for d in (0, 1):
                    f[(i, j, k, d)] = plsc.load_gather(dense, [dv + d])
            orow = (iota + b) * 32
            for d in (0, 1):
                g00 = f[(0, 0, 0, d)] + wz * (f[(0, 0, 1, d)] - f[(0, 0, 0, d)])
                g01 = f[(0, 1, 0, d)] + wz * (f[(0, 1, 1, d)] - f[(0, 1, 0, d)])
                g10 = f[(1, 0, 0, d)] + wz * (f[(1, 0, 1, d)] - f[(1, 0, 0, d)])
                g11 = f[(1, 1, 0, d)] + wz * (f[(1, 1, 1, d)] - f[(1, 1, 0, d)])
                h0 = g00 + wy * (g01 - g00)
                h1 = g10 + wy * (g11 - g10)
                r = h0 + wx * (h1 - h0)
                plsc.store_scatter(ob, [orow + (l * 2 + d)], r)
            return carry

        lax.fori_loop(0, VPC, vbody, 0)

    SLV = list(range(N_DENSE, N_LEVELS))

    def chunk_body(ch, carry):
        gb = wid * PPW + ch * C
        pltpu.sync_copy(xt0.at[pl.ds(gb, C)], xb0)
        pltpu.sync_copy(xt1.at[pl.ds(gb, C)], xb1)
        pltpu.sync_copy(xt2.at[pl.ds(gb, C)], xb2)
        pass1(SLV[0], wbufs[0], ibufs[0], cbufs[0])
        fire(0)
        for l in range(N_DENSE):
            dense_level(l)
        for t, l in enumerate(SLV):
            p = t % 2
            if t + 1 < len(SLV):
                pn = (t + 1) % 2
                pass1(SLV[t + 1], wbufs[pn], ibufs[pn], cbufs[pn])
                fire(pn)
            drain(p)
            pass2(l, wbufs[p], cbufs[p], rbufs[p])
        pltpu.sync_copy(ob, out.at[pl.ds(gb * 32, C * 32)])
        return carry

    lax.fori_loop(0, NCHUNK, chunk_body, 0)


def kernel(x, features, resolution):
    del resolution
    ori_shape = x.shape[:-1]
    xt = x.reshape(-1, 3).T
    featv = jnp.transpose(
        features.reshape(N_LEVELS, T // 128, 128, 2), (0, 1, 3, 2)
    ).reshape(NWORDS)
    out, _ = _hashgrid(xt[0], xt[1], xt[2], featv)
    return out.reshape(*ori_shape, 2 * N_LEVELS)
